# baseline (speedup 1.0000x reference)
import jax
import jax.numpy as jnp
from jax import lax
from jax.experimental import pallas as pl
from jax.experimental.pallas import tpu as pltpu

N_MATS = 4
B = 16384
D = 512
NUM_CLASSES = 10
ROWS = 2048
NB = B // ROWS
NSTEPS = N_MATS * NB


def _body(t_col_ref, x_ref, out_ref, acc_ref, oh_ref):
    j = pl.program_id(0)
    i = pl.program_id(1)
    step = j * N_MATS + i

    @pl.when(step == 0)
    def _init():
        acc_ref[...] = jnp.zeros((8, 128), dtype=jnp.float32)

    @pl.when(i == 0)
    def _mkoh():
        t = t_col_ref[...]
        oh = (t == lax.broadcasted_iota(jnp.int32, (ROWS, 128), 1)
              ).astype(jnp.float32)
        oh_ref[...] = oh
        acc_ref[1:2, :] = acc_ref[1:2, :] + jnp.sum(oh, axis=0, keepdims=True)

    x = x_ref[...]
    sq = x * x
    l = jnp.log2(sq + 1e-38)
    ones_col = jnp.ones((D, 1), dtype=jnp.float32)
    s1 = jnp.sum(sq, axis=1, keepdims=True)
    s3 = jnp.sum(x, axis=1, keepdims=True)
    s2 = lax.dot_general((x * l).astype(jnp.bfloat16),
                         ones_col.astype(jnp.bfloat16),
                         (((1,), (0,)), ((), ())),
                         preferred_element_type=jnp.float32)
    e = lax.rsqrt(s1) * (jnp.log2(s1) * s3 - s2)
    part = lax.dot_general(e, oh_ref[...], (((0,), (0,)), ((), ())),
                           preferred_element_type=jnp.float32)
    acc_ref[0:1, :] = acc_ref[0:1, :] + part

    @pl.when(step == NSTEPS - 1)
    def _fin():
        csum = acc_ref[0:1, :]
        cnt = acc_ref[1:2, :]
        present = cnt > 0
        c_present = jnp.sum(jnp.where(present, 1.0, 0.0))
        contrib = jnp.where(present, csum / jnp.where(present, cnt, 1.0), 0.0)
        total = jnp.sum(contrib) / (2.0 * N_MATS * c_present)
        out_ref[...] = jnp.full((1, 1), total, dtype=jnp.float32)


def kernel(ChannelNoiseMatixs, targets):
    targets = jnp.squeeze(targets)
    t_col = targets.reshape(B, 1)
    out = pl.pallas_call(
        _body,
        grid=(NB, N_MATS),
        in_specs=[
            pl.BlockSpec((ROWS, 1), lambda j, i: (j, 0)),
            pl.BlockSpec((ROWS, D), lambda j, i: (i * NB + j, 0)),
        ],
        out_specs=pl.BlockSpec((1, 1), lambda j, i: (0, 0)),
        out_shape=jax.ShapeDtypeStruct((1, 1), jnp.float32),
        scratch_shapes=[
            pltpu.VMEM((8, 128), jnp.float32),
            pltpu.VMEM((ROWS, 128), jnp.float32),
        ],
    )(t_col, ChannelNoiseMatixs.reshape(N_MATS * B, D))
    return out[0, 0]


# speedup vs baseline: 1.1555x; 1.1555x over previous
import jax
import jax.numpy as jnp
from jax import lax
from jax.experimental import pallas as pl
from jax.experimental.pallas import tpu as pltpu

N_MATS = 4
B = 16384
D = 512
NUM_CLASSES = 10
ROWS = 4096
NB = B // ROWS
NSTEPS = N_MATS * NB


def _body(t_col_ref, x_ref, out_ref, acc_ref, oh_ref):
    j = pl.program_id(0)
    i = pl.program_id(1)
    step = j * N_MATS + i

    @pl.when(step == 0)
    def _init():
        acc_ref[...] = jnp.zeros((8, 128), dtype=jnp.float32)

    @pl.when(i == 0)
    def _mkoh():
        t = t_col_ref[...]
        oh = (t == lax.broadcasted_iota(jnp.int32, (ROWS, 128), 1)
              ).astype(jnp.float32)
        oh_ref[...] = oh
        acc_ref[1:2, :] = acc_ref[1:2, :] + jnp.sum(oh, axis=0, keepdims=True)

    x = x_ref[...]
    sq = x * x
    l = jnp.log2(sq + 1e-38)
    ones_col = jnp.ones((D, 1), dtype=jnp.float32)
    s1 = lax.dot_general(sq, ones_col, (((1,), (0,)), ((), ())),
                         preferred_element_type=jnp.float32)
    s3 = jnp.sum(x, axis=1, keepdims=True)
    s2 = lax.dot_general(x * l, ones_col, (((1,), (0,)), ((), ())),
                         preferred_element_type=jnp.float32)
    e = lax.rsqrt(s1) * (jnp.log2(s1) * s3 - s2)
    part = lax.dot_general(e, oh_ref[...], (((0,), (0,)), ((), ())),
                           preferred_element_type=jnp.float32)
    acc_ref[0:1, :] = acc_ref[0:1, :] + part

    @pl.when(step == NSTEPS - 1)
    def _fin():
        csum = acc_ref[0:1, :]
        cnt = acc_ref[1:2, :]
        present = cnt > 0
        c_present = jnp.sum(jnp.where(present, 1.0, 0.0))
        contrib = jnp.where(present, csum / jnp.where(present, cnt, 1.0), 0.0)
        total = jnp.sum(contrib) / (2.0 * N_MATS * c_present)
        out_ref[...] = jnp.full((1, 1), total, dtype=jnp.float32)


def kernel(ChannelNoiseMatixs, targets):
    targets = jnp.squeeze(targets)
    t_col = targets.reshape(B, 1)
    out = pl.pallas_call(
        _body,
        grid=(NB, N_MATS),
        in_specs=[
            pl.BlockSpec((ROWS, 1), lambda j, i: (j, 0)),
            pl.BlockSpec((ROWS, D), lambda j, i: (i * NB + j, 0)),
        ],
        out_specs=pl.BlockSpec((1, 1), lambda j, i: (0, 0)),
        out_shape=jax.ShapeDtypeStruct((1, 1), jnp.float32),
        scratch_shapes=[
            pltpu.VMEM((8, 128), jnp.float32),
            pltpu.VMEM((ROWS, 128), jnp.float32),
        ],
    )(t_col, ChannelNoiseMatixs.reshape(N_MATS * B, D))
    return out[0, 0]
